# baseline (speedup 1.0000x reference)
import functools

import jax
import jax.numpy as jnp
from jax import lax
from jax.experimental import pallas as pl
from jax.experimental.pallas import tpu as pltpu
from jax.experimental.pallas import tpu_sc as plsc

B = 16384
EMB = 16
N_DENSE = 13

NC, NS = 2, 16
NW = NC * NS
BPW = B // NW
EPW = BPW * EMB
CHUNK = 128
NCHUNK = EPW // CHUNK
PR = B // 8
PRW = BPW // 8


def _sc_gather_body(ebf, elf, esf, etf, i0, i1, i2, i3,
                    out, idx_v, rows_v, sem):
    wid = lax.axis_index("s") * NC + lax.axis_index("c")
    ebase = wid * EPW
    pltpu.sync_copy(i0.at[pl.ds(ebase, EPW)], idx_v.at[0])
    pltpu.sync_copy(i1.at[pl.ds(ebase, EPW)], idx_v.at[1])
    pltpu.sync_copy(i2.at[pl.ds(ebase, EPW)], idx_v.at[2])
    pltpu.sync_copy(i3.at[pl.ds(ebase, EPW)], idx_v.at[3])
    tables = (ebf, elf, esf, etf)

    def step(g, carry):
        copies = []
        for f in range(4):
            for j in range(4):
                off = (g * 4 + j) * CHUNK
                copies.append(pltpu.async_copy(
                    tables[f].at[idx_v.at[f, pl.ds(off, CHUNK)]],
                    rows_v.at[f, pl.ds(off, CHUNK)],
                    sem))
        for c in copies:
            c.wait()
        return carry

    lax.fori_loop(0, NCHUNK // 4, step, 0, unroll=False)
    for f in range(4):
        pltpu.sync_copy(rows_v.at[f], out.at[f, pl.ds(ebase, EPW)])


def _sc_gather(ebf, elf, esf, etf, i0, i1, i2, i3):
    mesh = plsc.VectorSubcoreMesh(core_axis_name="c", subcore_axis_name="s")
    return pl.kernel(
        _sc_gather_body,
        mesh=mesh,
        out_type=jax.ShapeDtypeStruct((4, B * EMB), jnp.float32),
        scratch_types=[
            pltpu.VMEM((4, EPW), jnp.int32),
            pltpu.VMEM((4, EPW), jnp.float32),
            pltpu.SemaphoreType.DMA,
        ],
        compiler_params=pltpu.CompilerParams(use_tc_tiling_on_sc=False),
    )(ebf, elf, esf, etf, i0, i1, i2, i3)


def _tc_dense_body(emb_ref, dnp_ref, tgt_ref,
                   w1f_ref, w1d_ref, b1_ref, w2_ref, b2_ref, w3_ref,
                   sel_ref, pred_ref, loss_ref):
    e0 = emb_ref[0]
    e1 = emb_ref[1]
    e2 = emb_ref[2]
    e3 = emb_ref[3]
    s = e0 + e1 + e2 + e3
    sq = e0 * e0 + e1 * e1 + e2 * e2 + e3 * e3
    cross = 0.5 * jnp.dot(s * s - sq, sel_ref[...],
                          preferred_element_type=jnp.float32)
    h = (jnp.dot(e0, w1f_ref[0], preferred_element_type=jnp.float32)
         + jnp.dot(e1, w1f_ref[1], preferred_element_type=jnp.float32)
         + jnp.dot(e2, w1f_ref[2], preferred_element_type=jnp.float32)
         + jnp.dot(e3, w1f_ref[3], preferred_element_type=jnp.float32)
         + jnp.dot(dnp_ref[...], w1d_ref[...],
                   preferred_element_type=jnp.float32)
         + b1_ref[...][None, :])
    h = jnp.maximum(h, 0.0)
    h = jnp.dot(h, w2_ref[...], preferred_element_type=jnp.float32) \
        + b2_ref[...][None, :]
    h = jnp.maximum(h, 0.0)
    logit = jnp.dot(h, w3_ref[...],
                    preferred_element_type=jnp.float32) + cross
    pred = 1.0 / (1.0 + jnp.exp(-logit))
    pred_ref[...] = pred
    p = jnp.clip(pred, 1e-7, 1.0 - 1e-7)
    t = tgt_ref[...]
    loss_ref[...] = (-jnp.sum(
        t * jnp.log(p) + (1.0 - t) * jnp.log(1.0 - p))
        * (1.0 / B)).reshape(1, 1)


def _tc_dense(emb, dnp, tgt2, w1f, w1d, b1bd, w2bd, b2bd, w3bd, sel):
    return pl.pallas_call(
        _tc_dense_body,
        out_shape=[
            jax.ShapeDtypeStruct((PR, 8), jnp.float32),
            jax.ShapeDtypeStruct((1, 1), jnp.float32),
        ],
    )(emb, dnp, tgt2, w1f, w1d, b1bd, w2bd, b2bd, w3bd, sel)


def kernel(base_cd, level, sex, tag, dense_features, target,
           E_base, E_level, E_sex, E_tag, W1, b1, W2, b2, W3):
    d16 = jnp.arange(EMB, dtype=jnp.int32)[None, :]
    i0 = (base_cd.astype(jnp.int32)[:, None] * EMB + d16).reshape(-1)
    i1 = (level.astype(jnp.int32)[:, None] * EMB + d16).reshape(-1)
    i2 = (sex.astype(jnp.int32)[:, None] * EMB + d16).reshape(-1)
    i3 = (tag.astype(jnp.int32)[:, None] * EMB + d16).reshape(-1)
    flat = _sc_gather(
        E_base.reshape(-1), E_level.reshape(-1), E_sex.reshape(-1),
        E_tag.reshape(-1), i0, i1, i2, i3)
    emb = flat.reshape(4, PR, 128)
    dnp = jnp.pad(dense_features,
                  ((0, 0), (0, EMB - N_DENSE))).reshape(PR, 128)
    eye8 = jnp.eye(8, dtype=jnp.float32)
    w1f = jnp.stack([
        jnp.kron(eye8, W1[0:16]),
        jnp.kron(eye8, W1[16:32]),
        jnp.kron(eye8, W1[32:48]),
        jnp.kron(eye8, W1[48:64]),
    ])
    w1d = jnp.kron(eye8, jnp.pad(W1[64:77], ((0, 3), (0, 0))))
    b1bd = jnp.tile(b1, 8)
    w2bd = jnp.kron(eye8, W2)
    b2bd = jnp.tile(b2, 8)
    w3bd = jnp.kron(eye8, W3)
    sel = jnp.kron(eye8, jnp.ones((EMB, 1), jnp.float32))
    tgt2 = target.reshape(PR, 8)
    pred_p, loss = _tc_dense(emb, dnp, tgt2, w1f, w1d, b1bd, w2bd, b2bd,
                             w3bd, sel)
    return (pred_p.reshape(B), loss[0, 0])


# speedup vs baseline: 1.2740x; 1.2740x over previous
import functools

import jax
import jax.numpy as jnp
from jax import lax
from jax.experimental import pallas as pl
from jax.experimental.pallas import tpu as pltpu
from jax.experimental.pallas import tpu_sc as plsc

B = 16384
EMB = 16
N_DENSE = 13

NC, NS = 2, 16
NW = NC * NS
BPW = B // NW
GROUPS = BPW // 16
PR = B // 8
PRW = BPW // 8
V_LEVEL = 100
V_SEX = 4


def _sc_gather_body(ebt, ett, elt, est, i0, i1, i2, i3,
                    out, idx_v, st_v, pk_v, lvl_v, sex_v, sem):
    wid = lax.axis_index("s") * NC + lax.axis_index("c")
    base = wid * BPW
    orow = wid * PRW
    pltpu.sync_copy(i0.at[pl.ds(base, BPW)], idx_v.at[0])
    pltpu.sync_copy(i3.at[pl.ds(base, BPW)], idx_v.at[1])
    pltpu.sync_copy(i1.at[pl.ds(base, BPW)], idx_v.at[2])
    pltpu.sync_copy(i2.at[pl.ds(base, BPW)], idx_v.at[3])
    pltpu.sync_copy(elt, lvl_v)
    pltpu.sync_copy(est, sex_v)

    lane = lax.iota(jnp.int32, 16)
    rowoff = lax.shift_right_logical(lane, 3)
    colbase = lax.mul(lax.bitwise_and(lane, 7), EMB)

    def step(g, carry):
        row16 = g * 2 + rowoff
        for t, f, tbl in ((0, 0, ebt), (1, 3, ett)):
            vec = idx_v[t, pl.ds(g * 16, 16)]
            aligned = lax.bitwise_and(vec, -8)
            copies = []
            for l in range(16):
                copies.append(pltpu.async_copy(
                    tbl.at[:, pl.ds(pl.multiple_of(aligned[l], 8), 8)],
                    st_v.at[t, :, pl.ds(l * 8, 8)],
                    sem))
            for c in copies:
                c.wait()
            pos = lax.mul(lane, 8) + lax.bitwise_and(vec, 7)
            for d in range(16):
                d_vec = jnp.full((16,), d, jnp.int32)
                plsc.store_scatter(
                    pk_v, [jnp.full((16,), f, jnp.int32), row16,
                           colbase + d],
                    plsc.load_gather(st_v.at[t], [d_vec, pos]))
        lv_idx = idx_v[2, pl.ds(g * 16, 16)]
        sx_idx = idx_v[3, pl.ds(g * 16, 16)]
        for d in range(16):
            d_vec = jnp.full((16,), d, jnp.int32)
            col16 = colbase + d
            plsc.store_scatter(
                pk_v, [jnp.full((16,), 1, jnp.int32), row16, col16],
                plsc.load_gather(lvl_v, [d_vec, lv_idx]))
            plsc.store_scatter(
                pk_v, [jnp.full((16,), 2, jnp.int32), row16, col16],
                plsc.load_gather(sex_v, [d_vec, sx_idx]))
        return carry

    lax.fori_loop(0, GROUPS, step, 0, unroll=False)
    for f in range(4):
        pltpu.sync_copy(pk_v.at[f], out.at[f, pl.ds(orow, PRW)])


def _sc_gather(ebt, ett, elt, est, i0, i1, i2, i3):
    mesh = plsc.VectorSubcoreMesh(core_axis_name="c", subcore_axis_name="s")
    return pl.kernel(
        _sc_gather_body,
        mesh=mesh,
        out_type=jax.ShapeDtypeStruct((4, PR, 128), jnp.float32),
        scratch_types=[
            pltpu.VMEM((4, BPW), jnp.int32),
            pltpu.VMEM((2, EMB, 128), jnp.float32),
            pltpu.VMEM((4, PRW, 128), jnp.float32),
            pltpu.VMEM((EMB, V_LEVEL), jnp.float32),
            pltpu.VMEM((EMB, V_SEX), jnp.float32),
            pltpu.SemaphoreType.DMA,
        ],
        compiler_params=pltpu.CompilerParams(use_tc_tiling_on_sc=False,
                                             needs_layout_passes=False),
    )(ebt, ett, elt, est, i0, i1, i2, i3)


def _tc_dense_body(emb_ref, dnp_ref, tgt_ref,
                   w1f_ref, w1d_ref, b1_ref, w2_ref, b2_ref, w3_ref,
                   sel_ref, pred_ref, loss_ref):
    e0 = emb_ref[0]
    e1 = emb_ref[1]
    e2 = emb_ref[2]
    e3 = emb_ref[3]
    s = e0 + e1 + e2 + e3
    sq = e0 * e0 + e1 * e1 + e2 * e2 + e3 * e3
    cross = 0.5 * jnp.dot(s * s - sq, sel_ref[...],
                          preferred_element_type=jnp.float32)
    h = (jnp.dot(e0, w1f_ref[0], preferred_element_type=jnp.float32)
         + jnp.dot(e1, w1f_ref[1], preferred_element_type=jnp.float32)
         + jnp.dot(e2, w1f_ref[2], preferred_element_type=jnp.float32)
         + jnp.dot(e3, w1f_ref[3], preferred_element_type=jnp.float32)
         + jnp.dot(dnp_ref[...], w1d_ref[...],
                   preferred_element_type=jnp.float32)
         + b1_ref[...][None, :])
    h = jnp.maximum(h, 0.0)
    h = jnp.dot(h, w2_ref[...], preferred_element_type=jnp.float32) \
        + b2_ref[...][None, :]
    h = jnp.maximum(h, 0.0)
    logit = jnp.dot(h, w3_ref[...],
                    preferred_element_type=jnp.float32) + cross
    pred = 1.0 / (1.0 + jnp.exp(-logit))
    pred_ref[...] = pred
    p = jnp.clip(pred, 1e-7, 1.0 - 1e-7)
    t = tgt_ref[...]
    loss_ref[...] = (-jnp.sum(
        t * jnp.log(p) + (1.0 - t) * jnp.log(1.0 - p))
        * (1.0 / B)).reshape(1, 1)


def _tc_dense(emb, dnp, tgt2, w1f, w1d, b1bd, w2bd, b2bd, w3bd, sel):
    return pl.pallas_call(
        _tc_dense_body,
        out_shape=[
            jax.ShapeDtypeStruct((PR, 8), jnp.float32),
            jax.ShapeDtypeStruct((1, 1), jnp.float32),
        ],
    )(emb, dnp, tgt2, w1f, w1d, b1bd, w2bd, b2bd, w3bd, sel)


def kernel(base_cd, level, sex, tag, dense_features, target,
           E_base, E_level, E_sex, E_tag, W1, b1, W2, b2, W3):
    emb = _sc_gather(
        E_base.T, E_tag.T, E_level.T, E_sex.T,
        base_cd.astype(jnp.int32), level.astype(jnp.int32),
        sex.astype(jnp.int32), tag.astype(jnp.int32))
    dnp = jnp.pad(dense_features,
                  ((0, 0), (0, EMB - N_DENSE))).reshape(PR, 128)
    eye8 = jnp.eye(8, dtype=jnp.float32)
    w1f = jnp.stack([
        jnp.kron(eye8, W1[0:16]),
        jnp.kron(eye8, W1[16:32]),
        jnp.kron(eye8, W1[32:48]),
        jnp.kron(eye8, W1[48:64]),
    ])
    w1d = jnp.kron(eye8, jnp.pad(W1[64:77], ((0, 3), (0, 0))))
    b1bd = jnp.tile(b1, 8)
    w2bd = jnp.kron(eye8, W2)
    b2bd = jnp.tile(b2, 8)
    w3bd = jnp.kron(eye8, W3)
    sel = jnp.kron(eye8, jnp.ones((EMB, 1), jnp.float32))
    tgt2 = target.reshape(PR, 8)
    pred_p, loss = _tc_dense(emb, dnp, tgt2, w1f, w1d, b1bd, w2bd, b2bd,
                             w3bd, sel)
    return (pred_p.reshape(B), loss[0, 0])


# speedup vs baseline: 3.3509x; 2.6303x over previous
import functools

import jax
import jax.numpy as jnp
from jax import lax
from jax.experimental import pallas as pl
from jax.experimental.pallas import tpu as pltpu
from jax.experimental.pallas import tpu_sc as plsc

B = 16384
EMB = 16
N_DENSE = 13

NC, NS = 2, 16
NW = NC * NS
BPW = B // NW
GROUPS = BPW // 16
PR = B // 8
PRW = BPW // 8
V_LEVEL = 100
V_SEX = 4
CHUNK = 128


def _sc_gather_body(eb128, et128, elt, est, v0, b0, v3, b3, i1, i2,
                    out, idx_v, st_v, pk_v, lvl_v, sex_v, sem):
    wid = lax.axis_index("s") * NC + lax.axis_index("c")
    base = wid * BPW
    orow = wid * PRW
    pltpu.sync_copy(v0.at[pl.ds(base, BPW)], idx_v.at[0])
    pltpu.sync_copy(b0.at[pl.ds(base, BPW)], idx_v.at[1])
    pltpu.sync_copy(v3.at[pl.ds(base, BPW)], idx_v.at[2])
    pltpu.sync_copy(b3.at[pl.ds(base, BPW)], idx_v.at[3])
    pltpu.sync_copy(i1.at[pl.ds(base, BPW)], idx_v.at[4])
    pltpu.sync_copy(i2.at[pl.ds(base, BPW)], idx_v.at[5])
    pltpu.sync_copy(elt, lvl_v)
    pltpu.sync_copy(est, sex_v)

    lane = lax.iota(jnp.int32, 16)
    rowoff = lax.shift_right_logical(lane, 3)
    colbase = lax.mul(lax.bitwise_and(lane, 7), EMB)

    rounds = [(0, c) for c in range(4)] + [(1, c) for c in range(4)]
    tbls = (eb128, et128)
    vrow = (0, 2)
    brow = (1, 3)
    fout = (0, 3)

    def fire(r):
        t, c = rounds[r]
        return pltpu.async_copy(
            tbls[t].at[idx_v.at[brow[t], pl.ds(c * CHUNK, CHUNK)]],
            st_v.at[r % 2], sem)

    def extract(r):
        t, c = rounds[r]
        st2d = st_v.at[r % 2]
        for g in range(8):
            i_vec = g * 16 + lane
            vv = idx_v[vrow[t], pl.ds(c * CHUNK + g * 16, 16)]
            qb = lax.mul(lax.bitwise_and(vv, 7), EMB)
            grow = (c * CHUNK + g * 16) // 8 + rowoff
            for d in range(16):
                plsc.store_scatter(
                    pk_v,
                    [jnp.full((16,), fout[t], jnp.int32), grow,
                     colbase + d],
                    plsc.load_gather(st2d, [i_vec, qb + d]))

    pending = [fire(0), fire(1)]
    for r in range(8):
        pending[r % 2].wait()
        extract(r)
        if r + 2 < 8:
            pending[r % 2] = fire(r + 2)

    for g in range(GROUPS):
        lv_idx = idx_v[4, pl.ds(g * 16, 16)]
        sx_idx = idx_v[5, pl.ds(g * 16, 16)]
        row16 = g * 2 + rowoff
        for d in range(16):
            d_vec = jnp.full((16,), d, jnp.int32)
            col16 = colbase + d
            plsc.store_scatter(
                pk_v, [jnp.full((16,), 1, jnp.int32), row16, col16],
                plsc.load_gather(lvl_v, [d_vec, lv_idx]))
            plsc.store_scatter(
                pk_v, [jnp.full((16,), 2, jnp.int32), row16, col16],
                plsc.load_gather(sex_v, [d_vec, sx_idx]))
    for f in range(4):
        pltpu.sync_copy(pk_v.at[f], out.at[f, pl.ds(orow, PRW)])


def _sc_gather(eb128, et128, elt, est, v0, b0, v3, b3, i1, i2):
    mesh = plsc.VectorSubcoreMesh(core_axis_name="c", subcore_axis_name="s")
    return pl.kernel(
        _sc_gather_body,
        mesh=mesh,
        out_type=jax.ShapeDtypeStruct((4, PR, 128), jnp.float32),
        scratch_types=[
            pltpu.VMEM((6, BPW), jnp.int32),
            pltpu.VMEM((2, CHUNK, 128), jnp.float32),
            pltpu.VMEM((4, PRW, 128), jnp.float32),
            pltpu.VMEM((EMB, V_LEVEL), jnp.float32),
            pltpu.VMEM((EMB, V_SEX), jnp.float32),
            pltpu.SemaphoreType.DMA,
        ],
        compiler_params=pltpu.CompilerParams(use_tc_tiling_on_sc=False,
                                             needs_layout_passes=False),
    )(eb128, et128, elt, est, v0, b0, v3, b3, i1, i2)


def _tc_dense_body(emb_ref, dnp_ref, tgt_ref,
                   w1f_ref, w1d_ref, b1_ref, w2_ref, b2_ref, w3_ref,
                   sel_ref, pred_ref, loss_ref):
    e0 = emb_ref[0]
    e1 = emb_ref[1]
    e2 = emb_ref[2]
    e3 = emb_ref[3]
    s = e0 + e1 + e2 + e3
    sq = e0 * e0 + e1 * e1 + e2 * e2 + e3 * e3
    cross = 0.5 * jnp.dot(s * s - sq, sel_ref[...],
                          preferred_element_type=jnp.float32)
    h = (jnp.dot(e0, w1f_ref[0], preferred_element_type=jnp.float32)
         + jnp.dot(e1, w1f_ref[1], preferred_element_type=jnp.float32)
         + jnp.dot(e2, w1f_ref[2], preferred_element_type=jnp.float32)
         + jnp.dot(e3, w1f_ref[3], preferred_element_type=jnp.float32)
         + jnp.dot(dnp_ref[...], w1d_ref[...],
                   preferred_element_type=jnp.float32)
         + b1_ref[...][None, :])
    h = jnp.maximum(h, 0.0)
    h = jnp.dot(h, w2_ref[...], preferred_element_type=jnp.float32) \
        + b2_ref[...][None, :]
    h = jnp.maximum(h, 0.0)
    logit = jnp.dot(h, w3_ref[...],
                    preferred_element_type=jnp.float32) + cross
    pred = 1.0 / (1.0 + jnp.exp(-logit))
    pred_ref[...] = pred
    p = jnp.clip(pred, 1e-7, 1.0 - 1e-7)
    t = tgt_ref[...]
    loss_ref[...] = (-jnp.sum(
        t * jnp.log(p) + (1.0 - t) * jnp.log(1.0 - p))
        * (1.0 / B)).reshape(1, 1)


def _tc_dense(emb, dnp, tgt2, w1f, w1d, b1bd, w2bd, b2bd, w3bd, sel):
    return pl.pallas_call(
        _tc_dense_body,
        out_shape=[
            jax.ShapeDtypeStruct((PR, 8), jnp.float32),
            jax.ShapeDtypeStruct((1, 1), jnp.float32),
        ],
    )(emb, dnp, tgt2, w1f, w1d, b1bd, w2bd, b2bd, w3bd, sel)


def kernel(base_cd, level, sex, tag, dense_features, target,
           E_base, E_level, E_sex, E_tag, W1, b1, W2, b2, W3):
    v0 = base_cd.astype(jnp.int32)
    v3 = tag.astype(jnp.int32)
    emb = _sc_gather(
        E_base.reshape(125000, 128), E_tag.reshape(12500, 128),
        E_level.T, E_sex.T,
        v0, v0 >> 3, v3, v3 >> 3,
        level.astype(jnp.int32), sex.astype(jnp.int32))
    dnp = jnp.pad(dense_features,
                  ((0, 0), (0, EMB - N_DENSE))).reshape(PR, 128)
    eye8 = jnp.eye(8, dtype=jnp.float32)
    w1f = jnp.stack([
        jnp.kron(eye8, W1[0:16]),
        jnp.kron(eye8, W1[16:32]),
        jnp.kron(eye8, W1[32:48]),
        jnp.kron(eye8, W1[48:64]),
    ])
    w1d = jnp.kron(eye8, jnp.pad(W1[64:77], ((0, 3), (0, 0))))
    b1bd = jnp.tile(b1, 8)
    w2bd = jnp.kron(eye8, W2)
    b2bd = jnp.tile(b2, 8)
    w3bd = jnp.kron(eye8, W3)
    sel = jnp.kron(eye8, jnp.ones((EMB, 1), jnp.float32))
    tgt2 = target.reshape(PR, 8)
    pred_p, loss = _tc_dense(emb, dnp, tgt2, w1f, w1d, b1bd, w2bd, b2bd,
                             w3bd, sel)
    return (pred_p.reshape(B), loss[0, 0])


# speedup vs baseline: 6.2382x; 1.8616x over previous
import functools

import jax
import jax.numpy as jnp
from jax import lax
from jax.experimental import pallas as pl
from jax.experimental.pallas import tpu as pltpu
from jax.experimental.pallas import tpu_sc as plsc

B = 16384
EMB = 16
N_DENSE = 13

NC, NS = 2, 16
NW = NC * NS
BPW = B // NW
GROUPS = BPW // 16
PR = B // 8
PRW = BPW // 8
V_LEVEL = 100
V_SEX = 4
CHUNK = 128


VB_FULL = 7812
VT_FULL = 781
OUT_B = 125000
OUT_T = 12504
BATCH = 4
NIT_B = (245 + BATCH - 1) // BATCH
NIT_T = (25 + BATCH - 1) // BATCH


def _sc_detile_body(ebt, ett, tb, tt, outb, outt,
                    st_v, pk_v, tl_v, sem_i0, sem_i1, sem_o0, sem_o1):
    wid = lax.axis_index("s") * NC + lax.axis_index("c")
    lane = lax.iota(jnp.int32, 16)
    rowoff = lax.shift_right_logical(lane, 3)
    colbase = lax.mul(lax.bitwise_and(lane, 7), EMB)
    sems_i = (sem_i0, sem_i1)
    sems_o = (sem_o0, sem_o1)

    def one_table(tbl, out, nfull, nit):
        nb = lax.div(nfull - 1 - wid, 32) + 1

        def fire(it, slot, j):
            bi = it * BATCH + j

            @pl.when(bi < nb)
            def _():
                b = wid + bi * 32
                off = pl.multiple_of(b * 128, 128)
                pltpu.async_copy(tbl.at[:, pl.ds(off, 128)],
                                 st_v.at[slot, j], sems_i[slot])

        def wait_in(it, slot, j):
            bi = it * BATCH + j

            @pl.when(bi < nb)
            def _():
                pltpu.make_async_copy(tbl.at[:, pl.ds(0, 128)],
                                      st_v.at[slot, j],
                                      sems_i[slot]).wait()

        def out_fire(it, slot, j):
            bi = it * BATCH + j

            @pl.when(bi < nb)
            def _():
                b = wid + bi * 32
                orow = pl.multiple_of(b * 16, 16)
                pltpu.async_copy(pk_v.at[slot, j],
                                 out.at[pl.ds(orow, 16)], sems_o[slot])

        def out_wait(it, slot, j):
            bi = it * BATCH + j

            @pl.when(bi < nb)
            def _():
                pltpu.make_async_copy(pk_v.at[slot, j],
                                      out.at[pl.ds(0, 16)],
                                      sems_o[slot]).wait()

        def process(it, slot):
            for j in range(BATCH):
                wait_in(it, slot, j)
                st2d = st_v.at[slot, j]
                for d in range(16):
                    vals = [st2d[d, pl.ds(s * 16, 16)] for s in range(8)]
                    for s in range(8):
                        plsc.store_scatter(
                            pk_v.at[slot, j],
                            [s * 2 + rowoff, colbase + d], vals[s])
                out_fire(it, slot, j)

        for j in range(BATCH):
            fire(0, 0, j)
        nit2 = (nit + 1) // 2

        def body(it, carry):
            b0 = 2 * it
            b1 = 2 * it + 1
            for j in range(BATCH):
                fire(b1, 1, j)

            @pl.when(it >= 1)
            def _():
                for j in range(BATCH):
                    out_wait(b0 - 2, 0, j)

            process(b0, 0)
            for j in range(BATCH):
                fire(b0 + 2, 0, j)

            @pl.when(it >= 1)
            def _():
                for j in range(BATCH):
                    out_wait(b1 - 2, 1, j)

            process(b1, 1)
            return carry

        lax.fori_loop(0, nit2, body, 0, unroll=False)
        for j in range(BATCH):
            out_wait(2 * nit2 - 2, 0, j)
            out_wait(2 * nit2 - 1, 1, j)

    one_table(ebt, outb, VB_FULL, NIT_B)
    one_table(ett, outt, VT_FULL, NIT_T)

    @pl.when(wid == 0)
    def _():
        pltpu.sync_copy(tb, tl_v)
        for s in range(4):
            vvec = s * 16 + lane
            for d in range(16):
                plsc.store_scatter(
                    pk_v.at[0, 0],
                    [s * 2 + rowoff, colbase + d],
                    plsc.load_gather(tl_v, [vvec, jnp.full((16,), d,
                                                           jnp.int32)]))
        pltpu.sync_copy(pk_v.at[0, 0].at[pl.ds(0, 8)],
                        outb.at[pl.ds(124992, 8)])

    @pl.when(wid == 1)
    def _():
        pltpu.sync_copy(tt, tl_v.at[pl.ds(0, 32)])
        for s in range(2):
            vvec = s * 16 + lane
            for d in range(16):
                plsc.store_scatter(
                    pk_v.at[0, 0],
                    [s * 2 + rowoff, colbase + d],
                    plsc.load_gather(tl_v.at[pl.ds(0, 32)],
                                     [vvec, jnp.full((16,), d, jnp.int32)]))
        pltpu.sync_copy(pk_v.at[0, 0].at[pl.ds(0, 8)],
                        outt.at[pl.ds(12496, 8)])


def _sc_detile(ebt, ett, tb, tt):
    mesh = plsc.VectorSubcoreMesh(core_axis_name="c", subcore_axis_name="s")
    return pl.kernel(
        _sc_detile_body,
        mesh=mesh,
        out_type=[
            jax.ShapeDtypeStruct((OUT_B, 128), jnp.float32),
            jax.ShapeDtypeStruct((OUT_T, 128), jnp.float32),
        ],
        scratch_types=[
            pltpu.VMEM((2, BATCH, EMB, 128), jnp.float32),
            pltpu.VMEM((2, BATCH, EMB, 128), jnp.float32),
            pltpu.VMEM((64, EMB), jnp.float32),
            pltpu.SemaphoreType.DMA,
            pltpu.SemaphoreType.DMA,
            pltpu.SemaphoreType.DMA,
            pltpu.SemaphoreType.DMA,
        ],
        compiler_params=pltpu.CompilerParams(use_tc_tiling_on_sc=True,
                                             needs_layout_passes=False),
    )(ebt, ett, tb, tt)


def _sc_gather_body(eb128, et128, elt, est, v0, b0, v3, b3, i1, i2,
                    out, idx_v, st_v, pk_v, lvl_v, sex_v, sem):
    wid = lax.axis_index("s") * NC + lax.axis_index("c")
    base = wid * BPW
    orow = wid * PRW
    pltpu.sync_copy(v0.at[pl.ds(base, BPW)], idx_v.at[0])
    pltpu.sync_copy(b0.at[pl.ds(base, BPW)], idx_v.at[1])
    pltpu.sync_copy(v3.at[pl.ds(base, BPW)], idx_v.at[2])
    pltpu.sync_copy(b3.at[pl.ds(base, BPW)], idx_v.at[3])
    pltpu.sync_copy(i1.at[pl.ds(base, BPW)], idx_v.at[4])
    pltpu.sync_copy(i2.at[pl.ds(base, BPW)], idx_v.at[5])
    pltpu.sync_copy(elt, lvl_v)
    pltpu.sync_copy(est, sex_v)

    lane = lax.iota(jnp.int32, 16)
    rowoff = lax.shift_right_logical(lane, 3)
    colbase = lax.mul(lax.bitwise_and(lane, 7), EMB)

    rounds = [(0, c) for c in range(4)] + [(1, c) for c in range(4)]
    tbls = (eb128, et128)
    vrow = (0, 2)
    brow = (1, 3)
    fout = (0, 3)

    def fire(r):
        t, c = rounds[r]
        return pltpu.async_copy(
            tbls[t].at[idx_v.at[brow[t], pl.ds(c * CHUNK, CHUNK)]],
            st_v.at[r % 2], sem)

    def extract(r):
        t, c = rounds[r]
        st2d = st_v.at[r % 2]
        for g in range(8):
            i_vec = g * 16 + lane
            vv = idx_v[vrow[t], pl.ds(c * CHUNK + g * 16, 16)]
            qb = lax.mul(lax.bitwise_and(vv, 7), EMB)
            grow = (c * CHUNK + g * 16) // 8 + rowoff
            for d in range(16):
                plsc.store_scatter(
                    pk_v,
                    [jnp.full((16,), fout[t], jnp.int32), grow,
                     colbase + d],
                    plsc.load_gather(st2d, [i_vec, qb + d]))

    pending = [fire(0), fire(1)]
    for r in range(8):
        pending[r % 2].wait()
        extract(r)
        if r + 2 < 8:
            pending[r % 2] = fire(r + 2)

    for g in range(GROUPS):
        lv_idx = idx_v[4, pl.ds(g * 16, 16)]
        sx_idx = idx_v[5, pl.ds(g * 16, 16)]
        row16 = g * 2 + rowoff
        for d in range(16):
            d_vec = jnp.full((16,), d, jnp.int32)
            col16 = colbase + d
            plsc.store_scatter(
                pk_v, [jnp.full((16,), 1, jnp.int32), row16, col16],
                plsc.load_gather(lvl_v, [d_vec, lv_idx]))
            plsc.store_scatter(
                pk_v, [jnp.full((16,), 2, jnp.int32), row16, col16],
                plsc.load_gather(sex_v, [d_vec, sx_idx]))
    for f in range(4):
        pltpu.sync_copy(pk_v.at[f], out.at[f, pl.ds(orow, PRW)])


def _sc_gather(eb128, et128, elt, est, v0, b0, v3, b3, i1, i2):
    mesh = plsc.VectorSubcoreMesh(core_axis_name="c", subcore_axis_name="s")
    return pl.kernel(
        _sc_gather_body,
        mesh=mesh,
        out_type=jax.ShapeDtypeStruct((4, PR, 128), jnp.float32),
        scratch_types=[
            pltpu.VMEM((6, BPW), jnp.int32),
            pltpu.VMEM((2, CHUNK, 128), jnp.float32),
            pltpu.VMEM((4, PRW, 128), jnp.float32),
            pltpu.VMEM((EMB, V_LEVEL), jnp.float32),
            pltpu.VMEM((EMB, V_SEX), jnp.float32),
            pltpu.SemaphoreType.DMA,
        ],
        compiler_params=pltpu.CompilerParams(use_tc_tiling_on_sc=False,
                                             needs_layout_passes=False),
    )(eb128, et128, elt, est, v0, b0, v3, b3, i1, i2)


def _tc_dense_body(emb_ref, dnp_ref, tgt_ref,
                   w1f_ref, w1d_ref, b1_ref, w2_ref, b2_ref, w3_ref,
                   sel_ref, pred_ref, loss_ref):
    e0 = emb_ref[0]
    e1 = emb_ref[1]
    e2 = emb_ref[2]
    e3 = emb_ref[3]
    s = e0 + e1 + e2 + e3
    sq = e0 * e0 + e1 * e1 + e2 * e2 + e3 * e3
    cross = 0.5 * jnp.dot(s * s - sq, sel_ref[...],
                          preferred_element_type=jnp.float32)
    h = (jnp.dot(e0, w1f_ref[0], preferred_element_type=jnp.float32)
         + jnp.dot(e1, w1f_ref[1], preferred_element_type=jnp.float32)
         + jnp.dot(e2, w1f_ref[2], preferred_element_type=jnp.float32)
         + jnp.dot(e3, w1f_ref[3], preferred_element_type=jnp.float32)
         + jnp.dot(dnp_ref[...], w1d_ref[...],
                   preferred_element_type=jnp.float32)
         + b1_ref[...][None, :])
    h = jnp.maximum(h, 0.0)
    h = jnp.dot(h, w2_ref[...], preferred_element_type=jnp.float32) \
        + b2_ref[...][None, :]
    h = jnp.maximum(h, 0.0)
    logit = jnp.dot(h, w3_ref[...],
                    preferred_element_type=jnp.float32) + cross
    pred = 1.0 / (1.0 + jnp.exp(-logit))
    pred_ref[...] = pred
    p = jnp.clip(pred, 1e-7, 1.0 - 1e-7)
    t = tgt_ref[...]
    loss_ref[...] = (-jnp.sum(
        t * jnp.log(p) + (1.0 - t) * jnp.log(1.0 - p))
        * (1.0 / B)).reshape(1, 1)


def _tc_dense(emb, dnp, tgt2, w1f, w1d, b1bd, w2bd, b2bd, w3bd, sel):
    return pl.pallas_call(
        _tc_dense_body,
        out_shape=[
            jax.ShapeDtypeStruct((PR, 8), jnp.float32),
            jax.ShapeDtypeStruct((1, 1), jnp.float32),
        ],
    )(emb, dnp, tgt2, w1f, w1d, b1bd, w2bd, b2bd, w3bd, sel)


def kernel(base_cd, level, sex, tag, dense_features, target,
           E_base, E_level, E_sex, E_tag, W1, b1, W2, b2, W3):
    v0 = base_cd.astype(jnp.int32)
    v3 = tag.astype(jnp.int32)
    eb128, et128 = _sc_detile(E_base.T, E_tag.T,
                              E_base[VB_FULL * 128:], E_tag[VT_FULL * 128:])
    emb = _sc_gather(
        eb128, et128,
        E_level.T, E_sex.T,
        v0, v0 >> 3, v3, v3 >> 3,
        level.astype(jnp.int32), sex.astype(jnp.int32))
    dnp = jnp.pad(dense_features,
                  ((0, 0), (0, EMB - N_DENSE))).reshape(PR, 128)
    eye8 = jnp.eye(8, dtype=jnp.float32)
    w1f = jnp.stack([
        jnp.kron(eye8, W1[0:16]),
        jnp.kron(eye8, W1[16:32]),
        jnp.kron(eye8, W1[32:48]),
        jnp.kron(eye8, W1[48:64]),
    ])
    w1d = jnp.kron(eye8, jnp.pad(W1[64:77], ((0, 3), (0, 0))))
    b1bd = jnp.tile(b1, 8)
    w2bd = jnp.kron(eye8, W2)
    b2bd = jnp.tile(b2, 8)
    w3bd = jnp.kron(eye8, W3)
    sel = jnp.kron(eye8, jnp.ones((EMB, 1), jnp.float32))
    tgt2 = target.reshape(PR, 8)
    pred_p, loss = _tc_dense(emb, dnp, tgt2, w1f, w1d, b1bd, w2bd, b2bd,
                             w3bd, sel)
    return (pred_p.reshape(B), loss[0, 0])
